# baseline (speedup 1.0000x reference)
import jax
import jax.numpy as jnp
from jax import lax
from jax.experimental import pallas as pl
from jax.experimental.pallas import tpu as pltpu

C = 64
N = 4096
TT = 12
K = int(N * 0.8)
INV_SQRT_N = 1.0 / 64.0
ROW_BLOCK = 256
TSUM_BLOCK = 2048


def _round_bf16(x):
    u = lax.bitcast_convert_type(x, jnp.int32)
    lsb = jnp.bitwise_and(lax.shift_right_logical(u, 16), 1)
    r = jnp.bitwise_and(u + 32767 + lsb, jnp.int32(-65536))
    return lax.bitcast_convert_type(r, jnp.float32)


def _round_bf16_pos_or_neg(x):
    u = lax.bitcast_convert_type(x, jnp.uint32)
    lsb = jnp.bitwise_and(lax.shift_right_logical(u, jnp.uint32(16)), jnp.uint32(1))
    r = jnp.bitwise_and(u + jnp.uint32(32767) + lsb, jnp.uint32(0xFFFF0000))
    return lax.bitcast_convert_type(r, jnp.float32)


def _tsum_kernel(x_ref, o_ref):
    v = x_ref[...]
    acc = v[:, 0:1]
    for t in range(1, TT):
        acc = acc + v[:, t:t + 1]
    o_ref[...] = acc.astype(jnp.bfloat16)


def _rows_kernel(xsr_ref, mem_ref, xsf_ref, w_ref, b_ref, o_ref):
    xsr = xsr_ref[...]
    dn = (((0,), (0,)), ((), ()))

    l1 = lax.dot_general(mem_ref[...], xsr, dn,
                         preferred_element_type=jnp.float32) * INV_SQRT_N
    l1 = jnp.maximum(l1, 0.0)
    m1 = jnp.max(l1, axis=0, keepdims=True)
    e1 = jnp.exp(l1 - m1)
    a1 = _round_bf16(e1 / jnp.sum(e1, axis=0, keepdims=True))

    l2 = lax.dot_general(xsf_ref[...], xsr, dn,
                         preferred_element_type=jnp.float32) * INV_SQRT_N
    l2 = jnp.maximum(l2, 0.0)
    m2 = jnp.max(l2, axis=0, keepdims=True)
    e2 = jnp.exp(l2 - m2)
    a2 = _round_bf16(e2 / jnp.sum(e2, axis=0, keepdims=True))

    w0 = w_ref[0]
    w1 = w_ref[1]
    b = b_ref[0]
    ft = (a1 * w0 + a2 * w1) + b
    f = ft.T
    mf = jnp.max(f, axis=-1, keepdims=True)
    ef = jnp.exp(f - mf)
    g = ef / jnp.sum(ef, axis=-1, keepdims=True)

    bits = lax.bitcast_convert_type(g, jnp.int32)
    hi0 = jnp.max(bits, axis=-1, keepdims=True)
    lo0 = jnp.zeros_like(hi0)

    def body(_, carry):
        lo, hi = carry
        mid = lo + lax.shift_right_logical(hi - lo + 1, 1)
        cnt = jnp.sum((bits >= mid).astype(jnp.int32), axis=-1, keepdims=True)
        pred = cnt >= K
        return jnp.where(pred, mid, lo), jnp.where(pred, hi, mid - 1)

    t, _ = lax.fori_loop(0, 31, body, (lo0, hi0))

    gt = bits > t
    eq = bits == t
    need = K - jnp.sum(gt.astype(jnp.int32), axis=-1, keepdims=True)

    idx = lax.broadcasted_iota(jnp.int32, eq.shape, 1)
    eq_i = eq.astype(jnp.int32)

    def body2(_, carry):
        lo, hi = carry
        mid = lo + lax.shift_right_logical(hi - lo, 1)
        cnt = jnp.sum(jnp.where(idx <= mid, eq_i, 0), axis=-1, keepdims=True)
        pred = cnt >= need
        return jnp.where(pred, lo, mid), jnp.where(pred, mid, hi)

    lo2 = jnp.full_like(need, -1)
    hi2 = jnp.full_like(need, N - 1)
    _, cutoff = lax.fori_loop(0, 12, body2, (lo2, hi2))

    keep = gt | (eq & (idx <= cutoff))
    o_ref[...] = jnp.where(keep, g, 0.0)


def kernel(x, memory, fc_w, fc_b):
    x2 = x.reshape(C * N, TT)
    xs_flat = pl.pallas_call(
        _tsum_kernel,
        grid=(C * N // TSUM_BLOCK,),
        in_specs=[pl.BlockSpec((TSUM_BLOCK, TT), lambda i: (i, 0))],
        out_specs=pl.BlockSpec((TSUM_BLOCK, 1), lambda i: (i, 0)),
        out_shape=jax.ShapeDtypeStruct((C * N, 1), jnp.bfloat16),
    )(x2)
    xs = xs_flat.reshape(C, N)

    out = pl.pallas_call(
        _rows_kernel,
        grid=(N // ROW_BLOCK,),
        in_specs=[
            pl.BlockSpec((C, ROW_BLOCK), lambda i: (0, i)),
            pl.BlockSpec((C, N), lambda i: (0, 0)),
            pl.BlockSpec((C, N), lambda i: (0, 0)),
            pl.BlockSpec(memory_space=pltpu.SMEM),
            pl.BlockSpec(memory_space=pltpu.SMEM),
        ],
        out_specs=pl.BlockSpec((ROW_BLOCK, N), lambda i: (i, 0)),
        out_shape=jax.ShapeDtypeStruct((N, N), jnp.float32),
    )(xs, memory.astype(jnp.bfloat16), xs,
      _round_bf16_pos_or_neg(fc_w.reshape(2)), fc_b.reshape(1))
    return out


# speedup vs baseline: 1.0433x; 1.0433x over previous
import jax
import jax.numpy as jnp
from jax import lax
from jax.experimental import pallas as pl
from jax.experimental.pallas import tpu as pltpu

C = 64
N = 4096
TT = 12
K = int(N * 0.8)
INV_SQRT_N = 1.0 / 64.0
ROW_BLOCK = 256
TSUM_BLOCK = 2048


def _round_bf16(x):
    u = lax.bitcast_convert_type(x, jnp.int32)
    lsb = jnp.bitwise_and(lax.shift_right_logical(u, 16), 1)
    r = jnp.bitwise_and(u + 32767 + lsb, jnp.int32(-65536))
    return lax.bitcast_convert_type(r, jnp.float32)


def _round_bf16_pos_or_neg(x):
    u = lax.bitcast_convert_type(x, jnp.uint32)
    lsb = jnp.bitwise_and(lax.shift_right_logical(u, jnp.uint32(16)), jnp.uint32(1))
    r = jnp.bitwise_and(u + jnp.uint32(32767) + lsb, jnp.uint32(0xFFFF0000))
    return lax.bitcast_convert_type(r, jnp.float32)


def _tsum_kernel(x_ref, o_ref):
    v = x_ref[...]
    acc = v[:, 0:1]
    for t in range(1, TT):
        acc = acc + v[:, t:t + 1]
    o_ref[...] = acc.astype(jnp.bfloat16)


def _rows_kernel(xsr_ref, mem_ref, xsf_ref, w_ref, b_ref, o_ref):
    xsr = xsr_ref[...]
    dn = (((0,), (0,)), ((), ()))

    l1 = lax.dot_general(mem_ref[...], xsr, dn,
                         preferred_element_type=jnp.float32) * INV_SQRT_N
    l1 = jnp.maximum(l1, 0.0)
    m1 = jnp.max(l1, axis=0, keepdims=True)
    e1 = jnp.exp(l1 - m1)
    a1 = _round_bf16(e1 / jnp.sum(e1, axis=0, keepdims=True))

    l2 = lax.dot_general(xsf_ref[...], xsr, dn,
                         preferred_element_type=jnp.float32) * INV_SQRT_N
    l2 = jnp.maximum(l2, 0.0)
    m2 = jnp.max(l2, axis=0, keepdims=True)
    e2 = jnp.exp(l2 - m2)
    a2 = _round_bf16(e2 / jnp.sum(e2, axis=0, keepdims=True))

    w0 = w_ref[0]
    w1 = w_ref[1]
    b = b_ref[0]
    ft = (a1 * w0 + a2 * w1) + b
    f = ft.T
    mf = jnp.max(f, axis=-1, keepdims=True)
    ef = jnp.exp(f - mf)
    g = ef / jnp.sum(ef, axis=-1, keepdims=True)

    bits = lax.bitcast_convert_type(g, jnp.int32)
    hi0 = jnp.max(bits, axis=-1, keepdims=True)
    lo0 = jnp.min(bits, axis=-1, keepdims=True)

    def body(_, carry):
        lo, hi = carry
        mid = lo + lax.shift_right_logical(hi - lo + 1, 1)
        cnt = jnp.sum((bits >= mid).astype(jnp.int32), axis=-1, keepdims=True)
        pred = cnt >= K
        return jnp.where(pred, mid, lo), jnp.where(pred, hi, mid - 1)

    t, _ = lax.fori_loop(0, 26, body, (lo0, hi0))

    gt = bits > t
    eq = bits == t
    need = K - jnp.sum(gt.astype(jnp.int32), axis=-1, keepdims=True)

    idx = lax.broadcasted_iota(jnp.int32, eq.shape, 1)
    eq_i = eq.astype(jnp.int32)

    def body2(_, carry):
        lo, hi = carry
        mid = lo + lax.shift_right_logical(hi - lo, 1)
        cnt = jnp.sum(jnp.where(idx <= mid, eq_i, 0), axis=-1, keepdims=True)
        pred = cnt >= need
        return jnp.where(pred, lo, mid), jnp.where(pred, mid, hi)

    lo2 = jnp.full_like(need, -1)
    hi2 = jnp.full_like(need, N - 1)
    _, cutoff = lax.fori_loop(0, 12, body2, (lo2, hi2))

    keep = gt | (eq & (idx <= cutoff))
    o_ref[...] = jnp.where(keep, g, 0.0)


def kernel(x, memory, fc_w, fc_b):
    x2 = x.reshape(C * N, TT)
    xs_flat = pl.pallas_call(
        _tsum_kernel,
        grid=(C * N // TSUM_BLOCK,),
        in_specs=[pl.BlockSpec((TSUM_BLOCK, TT), lambda i: (i, 0))],
        out_specs=pl.BlockSpec((TSUM_BLOCK, 1), lambda i: (i, 0)),
        out_shape=jax.ShapeDtypeStruct((C * N, 1), jnp.bfloat16),
    )(x2)
    xs = xs_flat.reshape(C, N)

    out = pl.pallas_call(
        _rows_kernel,
        grid=(N // ROW_BLOCK,),
        in_specs=[
            pl.BlockSpec((C, ROW_BLOCK), lambda i: (0, i)),
            pl.BlockSpec((C, N), lambda i: (0, 0)),
            pl.BlockSpec((C, N), lambda i: (0, 0)),
            pl.BlockSpec(memory_space=pltpu.SMEM),
            pl.BlockSpec(memory_space=pltpu.SMEM),
        ],
        out_specs=pl.BlockSpec((ROW_BLOCK, N), lambda i: (i, 0)),
        out_shape=jax.ShapeDtypeStruct((N, N), jnp.float32),
    )(xs, memory.astype(jnp.bfloat16), xs,
      _round_bf16_pos_or_neg(fc_w.reshape(2)), fc_b.reshape(1))
    return out
